# baseline (speedup 1.0000x reference)
import jax
import jax.numpy as jnp
from jax.experimental import pallas as pl
from jax.experimental.pallas import tpu as pltpu

_ROWS, _COLS = 16384, 4096
_NCHUNK = 8
_CROWS = _ROWS // _NCHUNK


def _dma_copy_patch(x_hbm, o_hbm, tile_v, sem0, sems, tsem):
    first = pltpu.make_async_copy(
        x_hbm.at[pl.ds(0, _CROWS), :], o_hbm.at[pl.ds(0, _CROWS), :], sem0
    )
    first.start()
    rest = []
    for k in range(1, _NCHUNK):
        cp = pltpu.make_async_copy(
            x_hbm.at[pl.ds(k * _CROWS, _CROWS), :],
            o_hbm.at[pl.ds(k * _CROWS, _CROWS), :],
            sems.at[k - 1],
        )
        cp.start()
        rest.append(cp)

    stage = pltpu.make_async_copy(x_hbm.at[pl.ds(0, 8), pl.ds(0, 128)], tile_v, tsem)
    stage.start()
    stage.wait()
    tile = tile_v[...]
    r = jax.lax.broadcasted_iota(jnp.int32, (8, 128), 0)
    c = jax.lax.broadcasted_iota(jnp.int32, (8, 128), 1)
    tile = jnp.where((r == 0) & (c == 0), 10.0, tile)
    tile = jnp.where((r == 0) & (c == 2), 30.0, tile)
    tile = jnp.where((r == 1) & (c == 0), 40.0, tile)
    tile = jnp.where((r == 1) & (c == 1), 20.0, tile)
    tile_v[...] = tile

    first.wait()
    patch = pltpu.make_async_copy(tile_v, o_hbm.at[pl.ds(0, 8), pl.ds(0, 128)], tsem)
    patch.start()
    for cp in rest:
        cp.wait()
    patch.wait()


def kernel(x):
    return pl.pallas_call(
        _dma_copy_patch,
        in_specs=[pl.BlockSpec(memory_space=pl.ANY)],
        out_specs=pl.BlockSpec(memory_space=pl.ANY),
        out_shape=jax.ShapeDtypeStruct((_ROWS, _COLS), jnp.float32),
        scratch_shapes=[
            pltpu.VMEM((8, 128), jnp.float32),
            pltpu.SemaphoreType.DMA,
            pltpu.SemaphoreType.DMA((_NCHUNK - 1,)),
            pltpu.SemaphoreType.DMA,
        ],
    )(x)


# speedup vs baseline: 49.0609x; 49.0609x over previous
import jax
import jax.numpy as jnp
from jax.experimental import pallas as pl

_ROWS, _COLS = 16384, 4096
_BLOCK = 512


def _copy_patch_kernel(x_ref, o_ref):
    o_ref[...] = x_ref[...]

    @pl.when(pl.program_id(0) == 0)
    def _patch():
        tile = o_ref[0:8, 0:128]
        r = jax.lax.broadcasted_iota(jnp.int32, (8, 128), 0)
        c = jax.lax.broadcasted_iota(jnp.int32, (8, 128), 1)
        tile = jnp.where((r == 0) & (c == 0), 10.0, tile)
        tile = jnp.where((r == 0) & (c == 2), 30.0, tile)
        tile = jnp.where((r == 1) & (c == 0), 40.0, tile)
        tile = jnp.where((r == 1) & (c == 1), 20.0, tile)
        o_ref[0:8, 0:128] = tile


def kernel(x):
    return pl.pallas_call(
        _copy_patch_kernel,
        grid=(_ROWS // _BLOCK,),
        in_specs=[pl.BlockSpec((_BLOCK, _COLS), lambda i: (i, 0))],
        out_specs=pl.BlockSpec((_BLOCK, _COLS), lambda i: (i, 0)),
        out_shape=jax.ShapeDtypeStruct((_ROWS, _COLS), jnp.float32),
    )(x)
